# baseline (speedup 1.0000x reference)
import functools

import jax
import jax.numpy as jnp
from jax import lax
from jax.experimental import pallas as pl
from jax.experimental.pallas import tpu as pltpu
from jax.experimental.pallas import tpu_sc as plsc

N = 10000
E = 160000
D = 256
DH = 128
SROW = 144
G16 = 16
BN = 1000
CB = 80
NCHUNK = E // CB
_GDN = lax.GatherDimensionNumbers(offset_dims=(), collapsed_slice_dims=(0,),
                                  start_index_map=(0,))


def _enc_body(with_pre, x_ref, w1_ref, b1_ref, w2_ref, b2_ref, *rest):
    if with_pre:
        wx_ref, bx_ref, out_ref, pre_ref = rest
    else:
        (out_ref,) = rest
    h = jnp.maximum(x_ref[...] * w1_ref[...] + b1_ref[...], 0.0)
    out = jnp.dot(h, w2_ref[...], preferred_element_type=jnp.float32) + b2_ref[...]
    out_ref[...] = out
    if with_pre:
        p = jnp.dot(out, wx_ref[...], preferred_element_type=jnp.float32) + bx_ref[...]
        pre_ref[0] = p[:, :DH]
        pre_ref[1] = p[:, DH:]


def _enc(x, w1, b1, w2, b2, pre_w=None):
    with_pre = pre_w is not None
    wspec = pl.BlockSpec((D, D), lambda i: (0, 0))
    vspec = pl.BlockSpec((1, D), lambda i: (0, 0))
    in_specs = [pl.BlockSpec((BN, 1), lambda i: (i, 0)), vspec, vspec, wspec, vspec]
    args = [x.reshape(N, 1), w1.reshape(1, D), b1.reshape(1, D), w2, b2.reshape(1, D)]
    out_shape = [jax.ShapeDtypeStruct((N, D), jnp.float32)]
    out_specs = [pl.BlockSpec((BN, D), lambda i: (i, 0))]
    if with_pre:
        in_specs += [wspec, vspec]
        args += [pre_w[0], pre_w[1].reshape(1, D)]
        out_shape.append(jax.ShapeDtypeStruct((2, N, DH), jnp.float32))
        out_specs.append(pl.BlockSpec((2, BN, DH), lambda i: (0, i, 0)))
    r = pl.pallas_call(
        functools.partial(_enc_body, with_pre),
        grid=(N // BN,),
        in_specs=in_specs,
        out_specs=out_specs,
        out_shape=out_shape,
    )(*args)
    return r if with_pre else r[0]


def _upd_body(with_pre, x_ref, ev_ref, x0_ref, wa_ref, wc_ref, bc_ref, b1_ref,
              w2_ref, b2_ref, *rest):
    if with_pre:
        wx_ref, bx_ref, out_ref, pre_ref = rest
    else:
        (out_ref,) = rest
    agg = jnp.concatenate([ev_ref[0, :, :DH], ev_ref[1, :, :DH]], axis=1)
    s = ev_ref[0, :, DH:DH + 1]
    h = jnp.dot(x_ref[...], wa_ref[...], preferred_element_type=jnp.float32)
    h += jnp.dot(agg, wc_ref[...], preferred_element_type=jnp.float32)
    h = jnp.maximum(h + s * bc_ref[...] + b1_ref[...], 0.0)
    out = jnp.dot(h, w2_ref[...], preferred_element_type=jnp.float32)
    out = jnp.maximum(out + b2_ref[...] + x0_ref[...], 0.0)
    out_ref[...] = out
    if with_pre:
        p = jnp.dot(out, wx_ref[...], preferred_element_type=jnp.float32) + bx_ref[...]
        pre_ref[0] = p[:, :DH]
        pre_ref[1] = p[:, DH:]


def _upd(x, ev, x0, wu1a, wc, bc, bu1, wu2, bu2, pre_w=None):
    with_pre = pre_w is not None
    wspec = pl.BlockSpec((D, D), lambda i: (0, 0))
    vspec = pl.BlockSpec((1, D), lambda i: (0, 0))
    nspec = pl.BlockSpec((BN, D), lambda i: (i, 0))
    in_specs = [nspec, pl.BlockSpec((2, BN, SROW), lambda i: (0, i, 0)), nspec,
                wspec, wspec, vspec, vspec, wspec, vspec]
    args = [x, ev, x0, wu1a, wc, bc.reshape(1, D), bu1.reshape(1, D), wu2,
            bu2.reshape(1, D)]
    out_shape = [jax.ShapeDtypeStruct((N, D), jnp.float32)]
    out_specs = [nspec]
    if with_pre:
        in_specs += [wspec, vspec]
        args += [pre_w[0], pre_w[1].reshape(1, D)]
        out_shape.append(jax.ShapeDtypeStruct((2, N, DH), jnp.float32))
        out_specs.append(pl.BlockSpec((2, BN, DH), lambda i: (0, i, 0)))
    r = pl.pallas_call(
        functools.partial(_upd_body, with_pre),
        grid=(N // BN,),
        in_specs=in_specs,
        out_specs=out_specs,
        out_shape=out_shape,
    )(*args)
    return r if with_pre else r[0]


def _wcomb_body(m_ref, w_ref, out_ref):
    out_ref[0] = jnp.dot(m_ref[0], w_ref[0], preferred_element_type=jnp.float32)


def _wcomb(ms, wu1bs):
    return pl.pallas_call(
        _wcomb_body,
        grid=(4,),
        in_specs=[pl.BlockSpec((1, 264, D), lambda i: (i, 0, 0)),
                  pl.BlockSpec((1, D, D), lambda i: (i, 0, 0))],
        out_specs=pl.BlockSpec((1, 264, D), lambda i: (i, 0, 0)),
        out_shape=jax.ShapeDtypeStruct((4, 264, D), jnp.float32),
    )(ms, wu1bs)


def _pool_body(xv_ref, xc_ref, bv_ref, bc_ref, sv_ref, sc_ref, cv_ref, cc_ref):
    i = pl.program_id(0)
    gi = lax.broadcasted_iota(jnp.int32, (G16, BN), 0).astype(jnp.float32)
    mv = (gi == bv_ref[0]).astype(jnp.float32)
    mc = (gi == bc_ref[0]).astype(jnp.float32)
    pv = jnp.dot(mv, xv_ref[...], preferred_element_type=jnp.float32)
    pc = jnp.dot(mc, xc_ref[...], preferred_element_type=jnp.float32)
    cv = jnp.broadcast_to(jnp.sum(mv, axis=1, keepdims=True), (G16, 128))
    cc = jnp.broadcast_to(jnp.sum(mc, axis=1, keepdims=True), (G16, 128))

    @pl.when(i == 0)
    def _():
        sv_ref[...] = pv
        sc_ref[...] = pc
        cv_ref[...] = cv
        cc_ref[...] = cc

    @pl.when(i > 0)
    def _():
        sv_ref[...] += pv
        sc_ref[...] += pc
        cv_ref[...] += cv
        cc_ref[...] += cc


def _pool(xv, xc, bv, bc):
    bspec = pl.BlockSpec((1, 1, BN), lambda i: (i, 0, 0))
    nspec = pl.BlockSpec((BN, D), lambda i: (i, 0))
    sspec = pl.BlockSpec((G16, D), lambda i: (0, 0))
    cspec = pl.BlockSpec((G16, 128), lambda i: (0, 0))
    return pl.pallas_call(
        _pool_body,
        grid=(N // BN,),
        in_specs=[nspec, nspec, bspec, bspec],
        out_specs=[sspec, sspec, cspec, cspec],
        out_shape=[jax.ShapeDtypeStruct((G16, D), jnp.float32),
                   jax.ShapeDtypeStruct((G16, D), jnp.float32),
                   jax.ShapeDtypeStruct((G16, 128), jnp.float32),
                   jax.ShapeDtypeStruct((G16, 128), jnp.float32)],
    )(xv, xc, bv.astype(jnp.float32).reshape(N // BN, 1, BN),
      bc.astype(jnp.float32).reshape(N // BN, 1, BN))


def _final_body(sv_ref, sc_ref, cv_ref, cc_ref, w1_ref, b1_ref, w2_ref, b2_ref,
                out_ref):
    pred = sv_ref[...] / jnp.maximum(cv_ref[:, :1], 1.0)
    pred += sc_ref[...] / jnp.maximum(cc_ref[:, :1], 1.0)
    h = jnp.maximum(
        jnp.dot(pred, w1_ref[...], preferred_element_type=jnp.float32)
        + b1_ref[...], 0.0)
    out_ref[...] = jnp.dot(h, w2_ref[...],
                           preferred_element_type=jnp.float32) + b2_ref[...]


def _final(sv, sc, cv, cc, w1, b1, w2, b2):
    return pl.pallas_call(
        _final_body,
        out_shape=jax.ShapeDtypeStruct((G16, D), jnp.float32),
    )(sv, sc, cv, cc, w1, b1.reshape(1, D), w2, b2.reshape(1, D))


def _sc_edge(p2, meta_i, meta_f, w_attr2):
    mesh = plsc.VectorSubcoreMesh(core_axis_name="c", subcore_axis_name="s")

    @functools.partial(
        pl.kernel,
        mesh=mesh,
        compiler_params=pltpu.CompilerParams(use_tc_tiling_on_sc=False),
        out_type=jax.ShapeDtypeStruct((2, N, SROW), jnp.float32),
        scratch_types=[
            pltpu.VMEM((2, CB), jnp.int32),
            pltpu.VMEM((2, CB), jnp.int32),
            pltpu.VMEM((2, CB), jnp.float32),
            pltpu.VMEM((2, CB), jnp.float32),
            pltpu.VMEM((CB,), jnp.int32),
            pltpu.VMEM((CB,), jnp.int32),
            pltpu.VMEM((CB,), jnp.int32),
            pltpu.VMEM((CB,), jnp.int32),
            pltpu.VMEM((CB, DH), jnp.float32),
            pltpu.VMEM((CB, DH), jnp.float32),
            pltpu.VMEM((CB, SROW), jnp.float32),
            pltpu.VMEM((DH,), jnp.float32),
            pltpu.VMEM_SHARED((N, SROW), jnp.float32),
            pltpu.SemaphoreType.DMA,
            pltpu.SemaphoreType.DMA,
        ],
    )
    def k(p2_h, mti_h, mtf_h, wa_h, out_h,
          m0, m1, f0, f1, g0, g1, d0, d1, r0, r1, out_v, wa_v, acc, s0, s1):
        cid = lax.axis_index("c")
        sid = lax.axis_index("s")
        pltpu.sync_copy(wa_h.at[cid], wa_v)
        mbuf = (m0, m1)
        fbuf = (f0, f1)
        gbuf = (g0, g1)
        dbuf = (d0, d1)
        rbuf = (r0, r1)
        sems = (s0, s1)
        off = cid * N
        lane = lax.iota(jnp.int32, 16)

        def issue(gch, b):
            c = sid + 16 * gch
            pltpu.sync_copy(mti_h.at[c], mbuf[b])
            pltpu.sync_copy(mtf_h.at[c], fbuf[b])
            for j in range(CB // 16):
                gbuf[b][pl.ds(16 * j, 16)] = (
                    mbuf[b][0, pl.ds(16 * j, 16)] + off)
                dbuf[b][pl.ds(16 * j, 16)] = mbuf[b][1, pl.ds(16 * j, 16)]
            pltpu.async_copy(p2_h.at[gbuf[b]], rbuf[b], sems[b])

        def wait_g(b):
            pltpu.make_async_copy(p2_h.at[gbuf[b]], rbuf[b], sems[b]).wait()

        w_regs = [wa_v[pl.ds(16 * kk, 16)] for kk in range(DH // 16)]

        def compute_scatter(b):
            @plsc.parallel_loop(0, CB // 16)
            def group(j):
                a16 = fbuf[b][0, pl.ds(16 * j, 16)]
                n16 = fbuf[b][1, pl.ds(16 * j, 16)]
                for li in range(16):
                    sel = jnp.full((16, 1), li, jnp.int32)
                    a = lax.gather(a16, sel, _GDN, (1,),
                                   mode=lax.GatherScatterMode.PROMISE_IN_BOUNDS)
                    n = lax.gather(n16, sel, _GDN, (1,),
                                   mode=lax.GatherScatterMode.PROMISE_IN_BOUNDS)
                    e = 16 * j + li
                    out_v[e, pl.ds(DH, 16)] = jnp.where(lane == 0, n, 0.0)
                    for kk in range(DH // 16):
                        v = rbuf[b][e, pl.ds(16 * kk, 16)]
                        out_v[e, pl.ds(16 * kk, 16)] = (
                            jnp.maximum(v + a * w_regs[kk], 0.0) * n)

            pltpu.sync_copy(out_v, acc.at[dbuf[b]], add=True)

        issue(0, 0)

        z16 = jnp.zeros((16,), jnp.float32)

        def zrow(r, _):
            for j in range(SROW // 16):
                out_v[r, pl.ds(16 * j, 16)] = z16
            return 0

        lax.fori_loop(0, CB, zrow, 0)
        rows_per_tile = N // 16
        zbase = sid * rows_per_tile
        for t in range(rows_per_tile // CB):
            pltpu.sync_copy(out_v, acc.at[pl.ds(zbase + CB * t, CB)])
        zrem = rows_per_tile % CB
        if zrem:
            pltpu.sync_copy(
                out_v.at[pl.ds(0, zrem)],
                acc.at[pl.ds(zbase + (rows_per_tile // CB) * CB, zrem)])
        plsc.subcore_barrier()

        npt = NCHUNK // 16

        def body(gg, _):
            g0c = 2 * gg
            issue(g0c + 1, 1)
            wait_g(0)
            compute_scatter(0)
            issue(g0c + 2, 0)
            wait_g(1)
            compute_scatter(1)
            return 0

        lax.fori_loop(0, (npt - 1) // 2, body, 0)
        wait_g(0)
        compute_scatter(0)

        plsc.subcore_barrier()
        for t in range(rows_per_tile // CB):
            rows = pl.ds(zbase + CB * t, CB)
            pltpu.sync_copy(acc.at[rows], out_h.at[cid, rows])
        if zrem:
            rows = pl.ds(zbase + (rows_per_tile // CB) * CB, zrem)
            pltpu.sync_copy(acc.at[rows], out_h.at[cid, rows])

    return k(p2, meta_i, meta_f, w_attr2)


def kernel(b, q, edge_index_v2c, edge_attr_v2c, norm_v2c, norm_c2v,
           batch_vals, batch_cons, num_graphs, params):
    p = params
    src = edge_index_v2c[0]
    dst = edge_index_v2c[1]
    attr = edge_attr_v2c[:, 0]

    convs = p['convs']
    ms, wu1bs = [], []
    for lp in convs:
        for dname in ('v2c', 'c2v'):
            (_, _), (wm2, bm2) = lp[dname]['msg']
            (wu1, _), (_, _) = lp[dname]['upd']
            ms.append(jnp.concatenate(
                [wm2, bm2[None, :], jnp.zeros((7, D), jnp.float32)], axis=0))
            wu1bs.append(wu1[D:])
    comb = _wcomb(jnp.stack(ms), jnp.stack(wu1bs))

    def dir_params(li, dname, ci):
        (wm1, bm1), _ = convs[li][dname]['msg']
        (wu1, bu1), (wu2, bu2) = convs[li][dname]['upd']
        return dict(
            pre_w=(wm1[:D], bm1),
            w_attr2=wm1[D].reshape(2, DH),
            wu1a=wu1[:D], bu1=bu1, wu2=wu2, bu2=bu2,
            wc=comb[ci, :D], bc=comb[ci, D],
        )

    (bw1, bb1), (bw2, bb2) = p['b_enc']
    (qw1, qb1), (qw2, qb2) = p['q_enc']

    cons0 = _enc(b, bw1[0], bb1, bw2, bb2)
    l0v = dir_params(0, 'v2c', 0)
    vals0, pv = _enc(q, qw1[0], qb1, qw2, qb2, pre_w=l0v['pre_w'])

    def pack2(a0, a1):
        return jnp.stack([a0, a1]).reshape(2, NCHUNK, CB).transpose(1, 0, 2)

    mi_v2c = pack2(src, dst)
    mi_c2v = pack2(dst, src)
    mf_v2c = pack2(attr, norm_v2c)
    mf_c2v = pack2(attr, norm_c2v)

    x_cons, x_vals = cons0, vals0
    for li in range(2):
        dv = dir_params(li, 'v2c', 2 * li)
        dc = dir_params(li, 'c2v', 2 * li + 1)
        ev = _sc_edge(pv.reshape(2 * N, DH), mi_v2c, mf_v2c, dv['w_attr2'])
        x_cons, pc = _upd(x_cons, ev, cons0, dv['wu1a'], dv['wc'], dv['bc'],
                          dv['bu1'], dv['wu2'], dv['bu2'], pre_w=dc['pre_w'])
        ec = _sc_edge(pc.reshape(2 * N, DH), mi_c2v, mf_c2v, dc['w_attr2'])
        if li == 0:
            nxt = dir_params(1, 'v2c', 2)
            x_vals, pv = _upd(x_vals, ec, vals0, dc['wu1a'], dc['wc'],
                              dc['bc'], dc['bu1'], dc['wu2'], dc['bu2'],
                              pre_w=nxt['pre_w'])
        else:
            x_vals = _upd(x_vals, ec, vals0, dc['wu1a'], dc['wc'], dc['bc'],
                          dc['bu1'], dc['wu2'], dc['bu2'])

    sv, sc_, cv, cc = _pool(x_vals, x_cons, batch_vals, batch_cons)
    (fw1, fb1), (fw2, fb2) = p['fc']
    return _final(sv, sc_, cv, cc, fw1, fb1, fw2, fb2)


# speedup vs baseline: 1.3431x; 1.3431x over previous
import functools

import jax
import jax.numpy as jnp
from jax import lax
from jax.experimental import pallas as pl
from jax.experimental.pallas import tpu as pltpu
from jax.experimental.pallas import tpu_sc as plsc

N = 10000
E = 160000
D = 256
DH = 128
SROW = 144
G16 = 16
BN = 1000
CB = 80
NCHUNK = E // CB
_GDN = lax.GatherDimensionNumbers(offset_dims=(), collapsed_slice_dims=(0,),
                                  start_index_map=(0,))


def _enc_body(with_pre, x_ref, w1_ref, b1_ref, w2_ref, b2_ref, *rest):
    if with_pre:
        wx_ref, bx_ref, out_ref, pre_ref = rest
    else:
        (out_ref,) = rest
    h = jnp.maximum(x_ref[...] * w1_ref[...] + b1_ref[...], 0.0)
    out = jnp.dot(h, w2_ref[...], preferred_element_type=jnp.float32) + b2_ref[...]
    out_ref[...] = out
    if with_pre:
        p = jnp.dot(out, wx_ref[...], preferred_element_type=jnp.float32) + bx_ref[...]
        pre_ref[0] = p[:, :DH]
        pre_ref[1] = p[:, DH:]


def _enc(x, w1, b1, w2, b2, pre_w=None):
    with_pre = pre_w is not None
    wspec = pl.BlockSpec((D, D), lambda i: (0, 0))
    vspec = pl.BlockSpec((1, D), lambda i: (0, 0))
    in_specs = [pl.BlockSpec((BN, 1), lambda i: (i, 0)), vspec, vspec, wspec, vspec]
    args = [x.reshape(N, 1), w1.reshape(1, D), b1.reshape(1, D), w2, b2.reshape(1, D)]
    out_shape = [jax.ShapeDtypeStruct((N, D), jnp.float32)]
    out_specs = [pl.BlockSpec((BN, D), lambda i: (i, 0))]
    if with_pre:
        in_specs += [wspec, vspec]
        args += [pre_w[0], pre_w[1].reshape(1, D)]
        out_shape.append(jax.ShapeDtypeStruct((2, N, DH), jnp.float32))
        out_specs.append(pl.BlockSpec((2, BN, DH), lambda i: (0, i, 0)))
    r = pl.pallas_call(
        functools.partial(_enc_body, with_pre),
        grid=(N // BN,),
        in_specs=in_specs,
        out_specs=out_specs,
        out_shape=out_shape,
    )(*args)
    return r if with_pre else r[0]


def _upd_body(with_pre, x_ref, ev_ref, x0_ref, wa_ref, wc_ref, bc_ref, b1_ref,
              w2_ref, b2_ref, *rest):
    if with_pre:
        wx_ref, bx_ref, out_ref, pre_ref = rest
    else:
        (out_ref,) = rest
    agg = jnp.concatenate([ev_ref[0, :, :DH], ev_ref[1, :, :DH]], axis=1)
    s = ev_ref[0, :, DH:DH + 1]
    h = jnp.dot(x_ref[...], wa_ref[...], preferred_element_type=jnp.float32)
    h += jnp.dot(agg, wc_ref[...], preferred_element_type=jnp.float32)
    h = jnp.maximum(h + s * bc_ref[...] + b1_ref[...], 0.0)
    out = jnp.dot(h, w2_ref[...], preferred_element_type=jnp.float32)
    out = jnp.maximum(out + b2_ref[...] + x0_ref[...], 0.0)
    out_ref[...] = out
    if with_pre:
        p = jnp.dot(out, wx_ref[...], preferred_element_type=jnp.float32) + bx_ref[...]
        pre_ref[0] = p[:, :DH]
        pre_ref[1] = p[:, DH:]


def _upd(x, ev, x0, wu1a, wc, bc, bu1, wu2, bu2, pre_w=None):
    with_pre = pre_w is not None
    wspec = pl.BlockSpec((D, D), lambda i: (0, 0))
    vspec = pl.BlockSpec((1, D), lambda i: (0, 0))
    nspec = pl.BlockSpec((BN, D), lambda i: (i, 0))
    in_specs = [nspec, pl.BlockSpec((2, BN, SROW), lambda i: (0, i, 0)), nspec,
                wspec, wspec, vspec, vspec, wspec, vspec]
    args = [x, ev, x0, wu1a, wc, bc.reshape(1, D), bu1.reshape(1, D), wu2,
            bu2.reshape(1, D)]
    out_shape = [jax.ShapeDtypeStruct((N, D), jnp.float32)]
    out_specs = [nspec]
    if with_pre:
        in_specs += [wspec, vspec]
        args += [pre_w[0], pre_w[1].reshape(1, D)]
        out_shape.append(jax.ShapeDtypeStruct((2, N, DH), jnp.float32))
        out_specs.append(pl.BlockSpec((2, BN, DH), lambda i: (0, i, 0)))
    r = pl.pallas_call(
        functools.partial(_upd_body, with_pre),
        grid=(N // BN,),
        in_specs=in_specs,
        out_specs=out_specs,
        out_shape=out_shape,
    )(*args)
    return r if with_pre else r[0]


def _wcomb_body(m_ref, w_ref, out_ref):
    out_ref[0] = jnp.dot(m_ref[0], w_ref[0], preferred_element_type=jnp.float32)


def _wcomb(ms, wu1bs):
    return pl.pallas_call(
        _wcomb_body,
        grid=(4,),
        in_specs=[pl.BlockSpec((1, 264, D), lambda i: (i, 0, 0)),
                  pl.BlockSpec((1, D, D), lambda i: (i, 0, 0))],
        out_specs=pl.BlockSpec((1, 264, D), lambda i: (i, 0, 0)),
        out_shape=jax.ShapeDtypeStruct((4, 264, D), jnp.float32),
    )(ms, wu1bs)


def _pool_body(xv_ref, xc_ref, bv_ref, bc_ref, sv_ref, sc_ref, cv_ref, cc_ref):
    i = pl.program_id(0)
    gi = lax.broadcasted_iota(jnp.int32, (G16, BN), 0).astype(jnp.float32)
    mv = (gi == bv_ref[0]).astype(jnp.float32)
    mc = (gi == bc_ref[0]).astype(jnp.float32)
    pv = jnp.dot(mv, xv_ref[...], preferred_element_type=jnp.float32)
    pc = jnp.dot(mc, xc_ref[...], preferred_element_type=jnp.float32)
    cv = jnp.broadcast_to(jnp.sum(mv, axis=1, keepdims=True), (G16, 128))
    cc = jnp.broadcast_to(jnp.sum(mc, axis=1, keepdims=True), (G16, 128))

    @pl.when(i == 0)
    def _():
        sv_ref[...] = pv
        sc_ref[...] = pc
        cv_ref[...] = cv
        cc_ref[...] = cc

    @pl.when(i > 0)
    def _():
        sv_ref[...] += pv
        sc_ref[...] += pc
        cv_ref[...] += cv
        cc_ref[...] += cc


def _pool(xv, xc, bv, bc):
    bspec = pl.BlockSpec((1, 1, BN), lambda i: (i, 0, 0))
    nspec = pl.BlockSpec((BN, D), lambda i: (i, 0))
    sspec = pl.BlockSpec((G16, D), lambda i: (0, 0))
    cspec = pl.BlockSpec((G16, 128), lambda i: (0, 0))
    return pl.pallas_call(
        _pool_body,
        grid=(N // BN,),
        in_specs=[nspec, nspec, bspec, bspec],
        out_specs=[sspec, sspec, cspec, cspec],
        out_shape=[jax.ShapeDtypeStruct((G16, D), jnp.float32),
                   jax.ShapeDtypeStruct((G16, D), jnp.float32),
                   jax.ShapeDtypeStruct((G16, 128), jnp.float32),
                   jax.ShapeDtypeStruct((G16, 128), jnp.float32)],
    )(xv, xc, bv.astype(jnp.float32).reshape(N // BN, 1, BN),
      bc.astype(jnp.float32).reshape(N // BN, 1, BN))


def _final_body(sv_ref, sc_ref, cv_ref, cc_ref, w1_ref, b1_ref, w2_ref, b2_ref,
                out_ref):
    pred = sv_ref[...] / jnp.maximum(cv_ref[:, :1], 1.0)
    pred += sc_ref[...] / jnp.maximum(cc_ref[:, :1], 1.0)
    h = jnp.maximum(
        jnp.dot(pred, w1_ref[...], preferred_element_type=jnp.float32)
        + b1_ref[...], 0.0)
    out_ref[...] = jnp.dot(h, w2_ref[...],
                           preferred_element_type=jnp.float32) + b2_ref[...]


def _final(sv, sc, cv, cc, w1, b1, w2, b2):
    return pl.pallas_call(
        _final_body,
        out_shape=jax.ShapeDtypeStruct((G16, D), jnp.float32),
    )(sv, sc, cv, cc, w1, b1.reshape(1, D), w2, b2.reshape(1, D))


def _sc_edge(p2, meta_i, meta_f, w_attr2):
    mesh = plsc.VectorSubcoreMesh(core_axis_name="c", subcore_axis_name="s")

    @functools.partial(
        pl.kernel,
        mesh=mesh,
        compiler_params=pltpu.CompilerParams(use_tc_tiling_on_sc=False),
        out_type=jax.ShapeDtypeStruct((2, N, SROW), jnp.float32),
        scratch_types=[
            pltpu.VMEM((2, CB), jnp.int32),
            pltpu.VMEM((2, CB), jnp.int32),
            pltpu.VMEM((2, CB), jnp.float32),
            pltpu.VMEM((2, CB), jnp.float32),
            pltpu.VMEM((CB,), jnp.int32),
            pltpu.VMEM((CB,), jnp.int32),
            pltpu.VMEM((CB,), jnp.int32),
            pltpu.VMEM((CB,), jnp.int32),
            pltpu.VMEM((CB, DH), jnp.float32),
            pltpu.VMEM((CB, DH), jnp.float32),
            pltpu.VMEM((CB, SROW), jnp.float32),
            pltpu.VMEM((DH,), jnp.float32),
            pltpu.VMEM_SHARED((N, SROW), jnp.float32),
            pltpu.SemaphoreType.DMA,
            pltpu.SemaphoreType.DMA,
        ],
    )
    def k(p2_h, mti_h, mtf_h, wa_h, out_h,
          m0, m1, f0, f1, g0, g1, d0, d1, r0, r1, out_v, wa_v, acc, s0, s1):
        cid = lax.axis_index("c")
        sid = lax.axis_index("s")
        pltpu.sync_copy(wa_h.at[cid], wa_v)
        mbuf = (m0, m1)
        fbuf = (f0, f1)
        gbuf = (g0, g1)
        dbuf = (d0, d1)
        rbuf = (r0, r1)
        sems = (s0, s1)
        off = cid * N
        lane = lax.iota(jnp.int32, 16)

        def issue(gch, b):
            c = sid + 16 * gch
            pltpu.sync_copy(mti_h.at[c], mbuf[b])
            pltpu.sync_copy(mtf_h.at[c], fbuf[b])
            for j in range(CB // 16):
                gbuf[b][pl.ds(16 * j, 16)] = (
                    mbuf[b][0, pl.ds(16 * j, 16)] + off)
                dbuf[b][pl.ds(16 * j, 16)] = mbuf[b][1, pl.ds(16 * j, 16)]
            pltpu.async_copy(p2_h.at[gbuf[b]], rbuf[b], sems[b])

        def wait_g(b):
            pltpu.make_async_copy(p2_h.at[gbuf[b]], rbuf[b], sems[b]).wait()

        w_regs = [wa_v[pl.ds(16 * kk, 16)] for kk in range(DH // 16)]

        def compute_scatter(b):
            def group(j, _):
                a16 = fbuf[b][0, pl.ds(16 * j, 16)]
                n16 = fbuf[b][1, pl.ds(16 * j, 16)]
                for li in range(16):
                    sel = jnp.full((16, 1), li, jnp.int32)
                    a = lax.gather(a16, sel, _GDN, (1,),
                                   mode=lax.GatherScatterMode.PROMISE_IN_BOUNDS)
                    n = lax.gather(n16, sel, _GDN, (1,),
                                   mode=lax.GatherScatterMode.PROMISE_IN_BOUNDS)
                    e = 16 * j + li
                    out_v[e, pl.ds(DH, 16)] = jnp.where(lane == 0, n, 0.0)
                    for kk in range(DH // 16):
                        v = rbuf[b][e, pl.ds(16 * kk, 16)]
                        out_v[e, pl.ds(16 * kk, 16)] = (
                            jnp.maximum(v + a * w_regs[kk], 0.0) * n)
                return 0

            lax.fori_loop(0, CB // 16, group, 0, unroll=CB // 16)
            pltpu.sync_copy(out_v, acc.at[dbuf[b]], add=True)

        issue(0, 0)

        z16 = jnp.zeros((16,), jnp.float32)

        def zrow(r, _):
            for j in range(SROW // 16):
                out_v[r, pl.ds(16 * j, 16)] = z16
            return 0

        lax.fori_loop(0, CB, zrow, 0)
        rows_per_tile = N // 16
        zbase = sid * rows_per_tile
        for t in range(rows_per_tile // CB):
            pltpu.sync_copy(out_v, acc.at[pl.ds(zbase + CB * t, CB)])
        zrem = rows_per_tile % CB
        if zrem:
            pltpu.sync_copy(
                out_v.at[pl.ds(0, zrem)],
                acc.at[pl.ds(zbase + (rows_per_tile // CB) * CB, zrem)])
        plsc.subcore_barrier()

        npt = NCHUNK // 16

        def body(gg, _):
            g0c = 2 * gg
            issue(g0c + 1, 1)
            wait_g(0)
            compute_scatter(0)
            issue(g0c + 2, 0)
            wait_g(1)
            compute_scatter(1)
            return 0

        lax.fori_loop(0, (npt - 1) // 2, body, 0)
        wait_g(0)
        compute_scatter(0)

        plsc.subcore_barrier()
        for t in range(rows_per_tile // CB):
            rows = pl.ds(zbase + CB * t, CB)
            pltpu.sync_copy(acc.at[rows], out_h.at[cid, rows])
        if zrem:
            rows = pl.ds(zbase + (rows_per_tile // CB) * CB, zrem)
            pltpu.sync_copy(acc.at[rows], out_h.at[cid, rows])

    return k(p2, meta_i, meta_f, w_attr2)


def kernel(b, q, edge_index_v2c, edge_attr_v2c, norm_v2c, norm_c2v,
           batch_vals, batch_cons, num_graphs, params):
    p = params
    src = edge_index_v2c[0]
    dst = edge_index_v2c[1]
    attr = edge_attr_v2c[:, 0]

    convs = p['convs']
    ms, wu1bs = [], []
    for lp in convs:
        for dname in ('v2c', 'c2v'):
            (_, _), (wm2, bm2) = lp[dname]['msg']
            (wu1, _), (_, _) = lp[dname]['upd']
            ms.append(jnp.concatenate(
                [wm2, bm2[None, :], jnp.zeros((7, D), jnp.float32)], axis=0))
            wu1bs.append(wu1[D:])
    comb = _wcomb(jnp.stack(ms), jnp.stack(wu1bs))

    def dir_params(li, dname, ci):
        (wm1, bm1), _ = convs[li][dname]['msg']
        (wu1, bu1), (wu2, bu2) = convs[li][dname]['upd']
        return dict(
            pre_w=(wm1[:D], bm1),
            w_attr2=wm1[D].reshape(2, DH),
            wu1a=wu1[:D], bu1=bu1, wu2=wu2, bu2=bu2,
            wc=comb[ci, :D], bc=comb[ci, D],
        )

    (bw1, bb1), (bw2, bb2) = p['b_enc']
    (qw1, qb1), (qw2, qb2) = p['q_enc']

    cons0 = _enc(b, bw1[0], bb1, bw2, bb2)
    l0v = dir_params(0, 'v2c', 0)
    vals0, pv = _enc(q, qw1[0], qb1, qw2, qb2, pre_w=l0v['pre_w'])

    def pack2(a0, a1):
        return jnp.stack([a0, a1]).reshape(2, NCHUNK, CB).transpose(1, 0, 2)

    mi_v2c = pack2(src, dst)
    mi_c2v = pack2(dst, src)
    mf_v2c = pack2(attr, norm_v2c)
    mf_c2v = pack2(attr, norm_c2v)

    x_cons, x_vals = cons0, vals0
    for li in range(2):
        dv = dir_params(li, 'v2c', 2 * li)
        dc = dir_params(li, 'c2v', 2 * li + 1)
        ev = _sc_edge(pv.reshape(2 * N, DH), mi_v2c, mf_v2c, dv['w_attr2'])
        x_cons, pc = _upd(x_cons, ev, cons0, dv['wu1a'], dv['wc'], dv['bc'],
                          dv['bu1'], dv['wu2'], dv['bu2'], pre_w=dc['pre_w'])
        ec = _sc_edge(pc.reshape(2 * N, DH), mi_c2v, mf_c2v, dc['w_attr2'])
        if li == 0:
            nxt = dir_params(1, 'v2c', 2)
            x_vals, pv = _upd(x_vals, ec, vals0, dc['wu1a'], dc['wc'],
                              dc['bc'], dc['bu1'], dc['wu2'], dc['bu2'],
                              pre_w=nxt['pre_w'])
        else:
            x_vals = _upd(x_vals, ec, vals0, dc['wu1a'], dc['wc'], dc['bc'],
                          dc['bu1'], dc['wu2'], dc['bu2'])

    sv, sc_, cv, cc = _pool(x_vals, x_cons, batch_vals, batch_cons)
    (fw1, fb1), (fw2, fb2) = p['fc']
    return _final(sv, sc_, cv, cc, fw1, fb1, fw2, fb2)


# speedup vs baseline: 1.4861x; 1.1065x over previous
import functools

import jax
import jax.numpy as jnp
from jax import lax
from jax.experimental import pallas as pl
from jax.experimental.pallas import tpu as pltpu
from jax.experimental.pallas import tpu_sc as plsc

N = 10000
E = 160000
D = 256
DH = 128
SROW = 144
G16 = 16
BN = 1000
CB = 80
NCHUNK = E // CB
_GDN = lax.GatherDimensionNumbers(offset_dims=(), collapsed_slice_dims=(0,),
                                  start_index_map=(0,))


def _enc_body(with_pre, x_ref, w1_ref, b1_ref, w2_ref, b2_ref, *rest):
    if with_pre:
        wx_ref, bx_ref, out_ref, pre_ref = rest
    else:
        (out_ref,) = rest
    h = jnp.maximum(x_ref[...] * w1_ref[...] + b1_ref[...], 0.0)
    out = jnp.dot(h, w2_ref[...], preferred_element_type=jnp.float32) + b2_ref[...]
    out_ref[...] = out
    if with_pre:
        p = jnp.dot(out, wx_ref[...], preferred_element_type=jnp.float32) + bx_ref[...]
        pre_ref[0] = p[:, :DH]
        pre_ref[1] = p[:, DH:]


def _enc(x, w1, b1, w2, b2, pre_w=None):
    with_pre = pre_w is not None
    wspec = pl.BlockSpec((D, D), lambda i: (0, 0))
    vspec = pl.BlockSpec((1, D), lambda i: (0, 0))
    in_specs = [pl.BlockSpec((BN, 1), lambda i: (i, 0)), vspec, vspec, wspec, vspec]
    args = [x.reshape(N, 1), w1.reshape(1, D), b1.reshape(1, D), w2, b2.reshape(1, D)]
    out_shape = [jax.ShapeDtypeStruct((N, D), jnp.float32)]
    out_specs = [pl.BlockSpec((BN, D), lambda i: (i, 0))]
    if with_pre:
        in_specs += [wspec, vspec]
        args += [pre_w[0], pre_w[1].reshape(1, D)]
        out_shape.append(jax.ShapeDtypeStruct((2, N, DH), jnp.float32))
        out_specs.append(pl.BlockSpec((2, BN, DH), lambda i: (0, i, 0)))
    r = pl.pallas_call(
        functools.partial(_enc_body, with_pre),
        grid=(N // BN,),
        in_specs=in_specs,
        out_specs=out_specs,
        out_shape=out_shape,
    )(*args)
    return r if with_pre else r[0]


def _upd_body(with_pre, x_ref, ev_ref, x0_ref, wa_ref, wc_ref, bc_ref, b1_ref,
              w2_ref, b2_ref, *rest):
    if with_pre:
        wx_ref, bx_ref, out_ref, pre_ref = rest
    else:
        (out_ref,) = rest
    agg = jnp.concatenate([ev_ref[0, :, :DH], ev_ref[1, :, :DH]], axis=1)
    s = ev_ref[0, :, DH:DH + 1]
    h = jnp.dot(x_ref[...], wa_ref[...], preferred_element_type=jnp.float32)
    h += jnp.dot(agg, wc_ref[...], preferred_element_type=jnp.float32)
    h = jnp.maximum(h + s * bc_ref[...] + b1_ref[...], 0.0)
    out = jnp.dot(h, w2_ref[...], preferred_element_type=jnp.float32)
    out = jnp.maximum(out + b2_ref[...] + x0_ref[...], 0.0)
    out_ref[...] = out
    if with_pre:
        p = jnp.dot(out, wx_ref[...], preferred_element_type=jnp.float32) + bx_ref[...]
        pre_ref[0] = p[:, :DH]
        pre_ref[1] = p[:, DH:]


def _upd(x, ev, x0, wu1a, wc, bc, bu1, wu2, bu2, pre_w=None):
    with_pre = pre_w is not None
    wspec = pl.BlockSpec((D, D), lambda i: (0, 0))
    vspec = pl.BlockSpec((1, D), lambda i: (0, 0))
    nspec = pl.BlockSpec((BN, D), lambda i: (i, 0))
    in_specs = [nspec, pl.BlockSpec((2, BN, SROW), lambda i: (0, i, 0)), nspec,
                wspec, wspec, vspec, vspec, wspec, vspec]
    args = [x, ev, x0, wu1a, wc, bc.reshape(1, D), bu1.reshape(1, D), wu2,
            bu2.reshape(1, D)]
    out_shape = [jax.ShapeDtypeStruct((N, D), jnp.float32)]
    out_specs = [nspec]
    if with_pre:
        in_specs += [wspec, vspec]
        args += [pre_w[0], pre_w[1].reshape(1, D)]
        out_shape.append(jax.ShapeDtypeStruct((2, N, DH), jnp.float32))
        out_specs.append(pl.BlockSpec((2, BN, DH), lambda i: (0, i, 0)))
    r = pl.pallas_call(
        functools.partial(_upd_body, with_pre),
        grid=(N // BN,),
        in_specs=in_specs,
        out_specs=out_specs,
        out_shape=out_shape,
    )(*args)
    return r if with_pre else r[0]


def _wcomb_body(m_ref, w_ref, out_ref):
    out_ref[0] = jnp.dot(m_ref[0], w_ref[0], preferred_element_type=jnp.float32)


def _wcomb(ms, wu1bs):
    return pl.pallas_call(
        _wcomb_body,
        grid=(4,),
        in_specs=[pl.BlockSpec((1, 264, D), lambda i: (i, 0, 0)),
                  pl.BlockSpec((1, D, D), lambda i: (i, 0, 0))],
        out_specs=pl.BlockSpec((1, 264, D), lambda i: (i, 0, 0)),
        out_shape=jax.ShapeDtypeStruct((4, 264, D), jnp.float32),
    )(ms, wu1bs)


def _pool_body(xv_ref, xc_ref, bv_ref, bc_ref, sv_ref, sc_ref, cv_ref, cc_ref):
    i = pl.program_id(0)
    gi = lax.broadcasted_iota(jnp.int32, (G16, BN), 0).astype(jnp.float32)
    mv = (gi == bv_ref[0]).astype(jnp.float32)
    mc = (gi == bc_ref[0]).astype(jnp.float32)
    pv = jnp.dot(mv, xv_ref[...], preferred_element_type=jnp.float32)
    pc = jnp.dot(mc, xc_ref[...], preferred_element_type=jnp.float32)
    cv = jnp.broadcast_to(jnp.sum(mv, axis=1, keepdims=True), (G16, 128))
    cc = jnp.broadcast_to(jnp.sum(mc, axis=1, keepdims=True), (G16, 128))

    @pl.when(i == 0)
    def _():
        sv_ref[...] = pv
        sc_ref[...] = pc
        cv_ref[...] = cv
        cc_ref[...] = cc

    @pl.when(i > 0)
    def _():
        sv_ref[...] += pv
        sc_ref[...] += pc
        cv_ref[...] += cv
        cc_ref[...] += cc


def _pool(xv, xc, bv, bc):
    bspec = pl.BlockSpec((1, 1, BN), lambda i: (i, 0, 0))
    nspec = pl.BlockSpec((BN, D), lambda i: (i, 0))
    sspec = pl.BlockSpec((G16, D), lambda i: (0, 0))
    cspec = pl.BlockSpec((G16, 128), lambda i: (0, 0))
    return pl.pallas_call(
        _pool_body,
        grid=(N // BN,),
        in_specs=[nspec, nspec, bspec, bspec],
        out_specs=[sspec, sspec, cspec, cspec],
        out_shape=[jax.ShapeDtypeStruct((G16, D), jnp.float32),
                   jax.ShapeDtypeStruct((G16, D), jnp.float32),
                   jax.ShapeDtypeStruct((G16, 128), jnp.float32),
                   jax.ShapeDtypeStruct((G16, 128), jnp.float32)],
    )(xv, xc, bv.astype(jnp.float32).reshape(N // BN, 1, BN),
      bc.astype(jnp.float32).reshape(N // BN, 1, BN))


def _final_body(sv_ref, sc_ref, cv_ref, cc_ref, w1_ref, b1_ref, w2_ref, b2_ref,
                out_ref):
    pred = sv_ref[...] / jnp.maximum(cv_ref[:, :1], 1.0)
    pred += sc_ref[...] / jnp.maximum(cc_ref[:, :1], 1.0)
    h = jnp.maximum(
        jnp.dot(pred, w1_ref[...], preferred_element_type=jnp.float32)
        + b1_ref[...], 0.0)
    out_ref[...] = jnp.dot(h, w2_ref[...],
                           preferred_element_type=jnp.float32) + b2_ref[...]


def _final(sv, sc, cv, cc, w1, b1, w2, b2):
    return pl.pallas_call(
        _final_body,
        out_shape=jax.ShapeDtypeStruct((G16, D), jnp.float32),
    )(sv, sc, cv, cc, w1, b1.reshape(1, D), w2, b2.reshape(1, D))


def _sc_edge(p2, meta, w_attr2):
    mesh = plsc.VectorSubcoreMesh(core_axis_name="c", subcore_axis_name="s")

    @functools.partial(
        pl.kernel,
        mesh=mesh,
        compiler_params=pltpu.CompilerParams(use_tc_tiling_on_sc=False),
        out_type=jax.ShapeDtypeStruct((2, N, SROW), jnp.float32),
        scratch_types=[
            pltpu.VMEM((4, CB), jnp.float32),
            pltpu.VMEM((4, CB), jnp.float32),
            pltpu.VMEM((CB,), jnp.int32),
            pltpu.VMEM((CB,), jnp.int32),
            pltpu.VMEM((CB,), jnp.int32),
            pltpu.VMEM((CB,), jnp.int32),
            pltpu.VMEM((CB, DH), jnp.float32),
            pltpu.VMEM((CB, DH), jnp.float32),
            pltpu.VMEM((CB, SROW), jnp.float32),
            pltpu.VMEM((DH,), jnp.float32),
            pltpu.VMEM_SHARED((N, SROW), jnp.float32),
            pltpu.SemaphoreType.DMA,
            pltpu.SemaphoreType.DMA,
        ],
    )
    def k(p2_h, mt_h, wa_h, out_h,
          m0, m1, g0, g1, d0, d1, r0, r1, out_v, wa_v, acc, s0, s1):
        cid = lax.axis_index("c")
        sid = lax.axis_index("s")
        pltpu.sync_copy(wa_h.at[cid], wa_v)
        mbuf = (m0, m1)
        gbuf = (g0, g1)
        dbuf = (d0, d1)
        rbuf = (r0, r1)
        sems = (s0, s1)
        off = cid * N
        lane = lax.iota(jnp.int32, 16)

        def issue(gch, b):
            c = sid + 16 * gch
            pltpu.sync_copy(mt_h.at[c], mbuf[b])
            for j in range(CB // 16):
                sl = pl.ds(16 * j, 16)
                gbuf[b][sl] = mbuf[b][0, sl].astype(jnp.int32) + off
                dbuf[b][sl] = mbuf[b][1, sl].astype(jnp.int32)
            pltpu.async_copy(p2_h.at[gbuf[b]], rbuf[b], sems[b])

        def wait_g(b):
            pltpu.make_async_copy(p2_h.at[gbuf[b]], rbuf[b], sems[b]).wait()

        w_regs = [wa_v[pl.ds(16 * kk, 16)] for kk in range(DH // 16)]

        def compute_scatter(b):
            def group(j, _):
                a16 = mbuf[b][2, pl.ds(16 * j, 16)]
                n16 = mbuf[b][3, pl.ds(16 * j, 16)]
                for li in range(16):
                    sel = jnp.full((16, 1), li, jnp.int32)
                    a = lax.gather(a16, sel, _GDN, (1,),
                                   mode=lax.GatherScatterMode.PROMISE_IN_BOUNDS)
                    n = lax.gather(n16, sel, _GDN, (1,),
                                   mode=lax.GatherScatterMode.PROMISE_IN_BOUNDS)
                    e = 16 * j + li
                    out_v[e, pl.ds(DH, 16)] = jnp.where(lane == 0, n, 0.0)
                    for kk in range(DH // 16):
                        v = rbuf[b][e, pl.ds(16 * kk, 16)]
                        out_v[e, pl.ds(16 * kk, 16)] = (
                            jnp.maximum(v + a * w_regs[kk], 0.0) * n)
                return 0

            lax.fori_loop(0, CB // 16, group, 0, unroll=CB // 16)
            pltpu.sync_copy(out_v, acc.at[dbuf[b]], add=True)

        issue(0, 0)

        z16 = jnp.zeros((16,), jnp.float32)

        def zrow(r, _):
            for j in range(SROW // 16):
                out_v[r, pl.ds(16 * j, 16)] = z16
            return 0

        lax.fori_loop(0, CB, zrow, 0)
        rows_per_tile = N // 16
        zbase = sid * rows_per_tile
        for t in range(rows_per_tile // CB):
            pltpu.sync_copy(out_v, acc.at[pl.ds(zbase + CB * t, CB)])
        zrem = rows_per_tile % CB
        if zrem:
            pltpu.sync_copy(
                out_v.at[pl.ds(0, zrem)],
                acc.at[pl.ds(zbase + (rows_per_tile // CB) * CB, zrem)])
        plsc.subcore_barrier()

        npt = NCHUNK // 16

        def body(gg, _):
            g0c = 2 * gg
            issue(g0c + 1, 1)
            wait_g(0)
            compute_scatter(0)
            issue(g0c + 2, 0)
            wait_g(1)
            compute_scatter(1)
            return 0

        lax.fori_loop(0, (npt - 1) // 2, body, 0)
        wait_g(0)
        compute_scatter(0)

        plsc.subcore_barrier()
        for t in range(rows_per_tile // CB):
            rows = pl.ds(zbase + CB * t, CB)
            pltpu.sync_copy(acc.at[rows], out_h.at[cid, rows])
        if zrem:
            rows = pl.ds(zbase + (rows_per_tile // CB) * CB, zrem)
            pltpu.sync_copy(acc.at[rows], out_h.at[cid, rows])

    return k(p2, meta, w_attr2)


def kernel(b, q, edge_index_v2c, edge_attr_v2c, norm_v2c, norm_c2v,
           batch_vals, batch_cons, num_graphs, params):
    p = params
    src = edge_index_v2c[0]
    dst = edge_index_v2c[1]
    attr = edge_attr_v2c[:, 0]

    convs = p['convs']
    ms, wu1bs = [], []
    for lp in convs:
        for dname in ('v2c', 'c2v'):
            (_, _), (wm2, bm2) = lp[dname]['msg']
            (wu1, _), (_, _) = lp[dname]['upd']
            ms.append(jnp.concatenate(
                [wm2, bm2[None, :], jnp.zeros((7, D), jnp.float32)], axis=0))
            wu1bs.append(wu1[D:])
    comb = _wcomb(jnp.stack(ms), jnp.stack(wu1bs))

    def dir_params(li, dname, ci):
        (wm1, bm1), _ = convs[li][dname]['msg']
        (wu1, bu1), (wu2, bu2) = convs[li][dname]['upd']
        return dict(
            pre_w=(wm1[:D], bm1),
            w_attr2=wm1[D].reshape(2, DH),
            wu1a=wu1[:D], bu1=bu1, wu2=wu2, bu2=bu2,
            wc=comb[ci, :D], bc=comb[ci, D],
        )

    (bw1, bb1), (bw2, bb2) = p['b_enc']
    (qw1, qb1), (qw2, qb2) = p['q_enc']

    cons0 = _enc(b, bw1[0], bb1, bw2, bb2)
    l0v = dir_params(0, 'v2c', 0)
    vals0, pv = _enc(q, qw1[0], qb1, qw2, qb2, pre_w=l0v['pre_w'])

    def pack4(a0, a1, a2, a3):
        m = jnp.stack([a0.astype(jnp.float32), a1.astype(jnp.float32), a2, a3])
        return m.reshape(4, NCHUNK, CB).transpose(1, 0, 2)

    mt_v2c = pack4(src, dst, attr, norm_v2c)
    mt_c2v = pack4(dst, src, attr, norm_c2v)

    x_cons, x_vals = cons0, vals0
    for li in range(2):
        dv = dir_params(li, 'v2c', 2 * li)
        dc = dir_params(li, 'c2v', 2 * li + 1)
        ev = _sc_edge(pv.reshape(2 * N, DH), mt_v2c, dv['w_attr2'])
        x_cons, pc = _upd(x_cons, ev, cons0, dv['wu1a'], dv['wc'], dv['bc'],
                          dv['bu1'], dv['wu2'], dv['bu2'], pre_w=dc['pre_w'])
        ec = _sc_edge(pc.reshape(2 * N, DH), mt_c2v, dc['w_attr2'])
        if li == 0:
            nxt = dir_params(1, 'v2c', 2)
            x_vals, pv = _upd(x_vals, ec, vals0, dc['wu1a'], dc['wc'],
                              dc['bc'], dc['bu1'], dc['wu2'], dc['bu2'],
                              pre_w=nxt['pre_w'])
        else:
            x_vals = _upd(x_vals, ec, vals0, dc['wu1a'], dc['wc'], dc['bc'],
                          dc['bu1'], dc['wu2'], dc['bu2'])

    sv, sc_, cv, cc = _pool(x_vals, x_cons, batch_vals, batch_cons)
    (fw1, fb1), (fw2, fb2) = p['fc']
    return _final(sv, sc_, cv, cc, fw1, fb1, fw2, fb2)


# speedup vs baseline: 1.5646x; 1.0528x over previous
import functools

import jax
import jax.numpy as jnp
from jax import lax
from jax.experimental import pallas as pl
from jax.experimental.pallas import tpu as pltpu
from jax.experimental.pallas import tpu_sc as plsc

N = 10000
E = 160000
D = 256
DH = 128
SROW = 144
G16 = 16
BN = 1000
CB = 80
NCHUNK = E // CB
_GDN = lax.GatherDimensionNumbers(offset_dims=(), collapsed_slice_dims=(0,),
                                  start_index_map=(0,))


def _enc_body(with_pre, x_ref, w1_ref, b1_ref, w2_ref, b2_ref, *rest):
    if with_pre:
        wx_ref, bx_ref, out_ref, pre_ref = rest
    else:
        (out_ref,) = rest
    h = jnp.maximum(x_ref[...] * w1_ref[...] + b1_ref[...], 0.0)
    out = jnp.dot(h, w2_ref[...], preferred_element_type=jnp.float32) + b2_ref[...]
    out_ref[...] = out
    if with_pre:
        p = jnp.dot(out, wx_ref[...], preferred_element_type=jnp.float32) + bx_ref[...]
        pre_ref[0] = p[:, :DH]
        pre_ref[1] = p[:, DH:]


def _enc(x, w1, b1, w2, b2, pre_w=None):
    with_pre = pre_w is not None
    wspec = pl.BlockSpec((D, D), lambda i: (0, 0))
    vspec = pl.BlockSpec((1, D), lambda i: (0, 0))
    in_specs = [pl.BlockSpec((BN, 1), lambda i: (i, 0)), vspec, vspec, wspec, vspec]
    args = [x.reshape(N, 1), w1.reshape(1, D), b1.reshape(1, D), w2, b2.reshape(1, D)]
    out_shape = [jax.ShapeDtypeStruct((N, D), jnp.float32)]
    out_specs = [pl.BlockSpec((BN, D), lambda i: (i, 0))]
    if with_pre:
        in_specs += [wspec, vspec]
        args += [pre_w[0], pre_w[1].reshape(1, D)]
        out_shape.append(jax.ShapeDtypeStruct((2, N, DH), jnp.float32))
        out_specs.append(pl.BlockSpec((2, BN, DH), lambda i: (0, i, 0)))
    r = pl.pallas_call(
        functools.partial(_enc_body, with_pre),
        grid=(N // BN,),
        in_specs=in_specs,
        out_specs=out_specs,
        out_shape=out_shape,
    )(*args)
    return r if with_pre else r[0]


def _upd_body(with_pre, x_ref, ev_ref, x0_ref, wa_ref, wc_ref, bc_ref, b1_ref,
              w2_ref, b2_ref, *rest):
    if with_pre:
        wx_ref, bx_ref, out_ref, pre_ref = rest
    else:
        (out_ref,) = rest
    agg = jnp.concatenate([ev_ref[0, :, :DH], ev_ref[1, :, :DH]], axis=1)
    s = ev_ref[0, :, DH:DH + 1]
    h = jnp.dot(x_ref[...], wa_ref[...], preferred_element_type=jnp.float32)
    h += jnp.dot(agg, wc_ref[...], preferred_element_type=jnp.float32)
    h = jnp.maximum(h + s * bc_ref[...] + b1_ref[...], 0.0)
    out = jnp.dot(h, w2_ref[...], preferred_element_type=jnp.float32)
    out = jnp.maximum(out + b2_ref[...] + x0_ref[...], 0.0)
    out_ref[...] = out
    if with_pre:
        p = jnp.dot(out, wx_ref[...], preferred_element_type=jnp.float32) + bx_ref[...]
        pre_ref[0] = p[:, :DH]
        pre_ref[1] = p[:, DH:]


def _upd(x, ev, x0, wu1a, wc, bc, bu1, wu2, bu2, pre_w=None):
    with_pre = pre_w is not None
    wspec = pl.BlockSpec((D, D), lambda i: (0, 0))
    vspec = pl.BlockSpec((1, D), lambda i: (0, 0))
    nspec = pl.BlockSpec((BN, D), lambda i: (i, 0))
    in_specs = [nspec, pl.BlockSpec((2, BN, SROW), lambda i: (0, i, 0)), nspec,
                wspec, wspec, vspec, vspec, wspec, vspec]
    args = [x, ev, x0, wu1a, wc, bc.reshape(1, D), bu1.reshape(1, D), wu2,
            bu2.reshape(1, D)]
    out_shape = [jax.ShapeDtypeStruct((N, D), jnp.float32)]
    out_specs = [nspec]
    if with_pre:
        in_specs += [wspec, vspec]
        args += [pre_w[0], pre_w[1].reshape(1, D)]
        out_shape.append(jax.ShapeDtypeStruct((2, N, DH), jnp.float32))
        out_specs.append(pl.BlockSpec((2, BN, DH), lambda i: (0, i, 0)))
    r = pl.pallas_call(
        functools.partial(_upd_body, with_pre),
        grid=(N // BN,),
        in_specs=in_specs,
        out_specs=out_specs,
        out_shape=out_shape,
    )(*args)
    return r if with_pre else r[0]


def _wcomb_body(m_ref, w_ref, out_ref):
    out_ref[0] = jnp.dot(m_ref[0], w_ref[0], preferred_element_type=jnp.float32)


def _wcomb(ms, wu1bs):
    return pl.pallas_call(
        _wcomb_body,
        grid=(4,),
        in_specs=[pl.BlockSpec((1, 264, D), lambda i: (i, 0, 0)),
                  pl.BlockSpec((1, D, D), lambda i: (i, 0, 0))],
        out_specs=pl.BlockSpec((1, 264, D), lambda i: (i, 0, 0)),
        out_shape=jax.ShapeDtypeStruct((4, 264, D), jnp.float32),
    )(ms, wu1bs)


def _pool_body(xv_ref, xc_ref, bv_ref, bc_ref, sv_ref, sc_ref, cv_ref, cc_ref):
    i = pl.program_id(0)
    gi = lax.broadcasted_iota(jnp.int32, (G16, BN), 0).astype(jnp.float32)
    mv = (gi == bv_ref[0]).astype(jnp.float32)
    mc = (gi == bc_ref[0]).astype(jnp.float32)
    pv = jnp.dot(mv, xv_ref[...], preferred_element_type=jnp.float32)
    pc = jnp.dot(mc, xc_ref[...], preferred_element_type=jnp.float32)
    cv = jnp.broadcast_to(jnp.sum(mv, axis=1, keepdims=True), (G16, 128))
    cc = jnp.broadcast_to(jnp.sum(mc, axis=1, keepdims=True), (G16, 128))

    @pl.when(i == 0)
    def _():
        sv_ref[...] = pv
        sc_ref[...] = pc
        cv_ref[...] = cv
        cc_ref[...] = cc

    @pl.when(i > 0)
    def _():
        sv_ref[...] += pv
        sc_ref[...] += pc
        cv_ref[...] += cv
        cc_ref[...] += cc


def _pool(xv, xc, bv, bc):
    bspec = pl.BlockSpec((1, 1, BN), lambda i: (i, 0, 0))
    nspec = pl.BlockSpec((BN, D), lambda i: (i, 0))
    sspec = pl.BlockSpec((G16, D), lambda i: (0, 0))
    cspec = pl.BlockSpec((G16, 128), lambda i: (0, 0))
    return pl.pallas_call(
        _pool_body,
        grid=(N // BN,),
        in_specs=[nspec, nspec, bspec, bspec],
        out_specs=[sspec, sspec, cspec, cspec],
        out_shape=[jax.ShapeDtypeStruct((G16, D), jnp.float32),
                   jax.ShapeDtypeStruct((G16, D), jnp.float32),
                   jax.ShapeDtypeStruct((G16, 128), jnp.float32),
                   jax.ShapeDtypeStruct((G16, 128), jnp.float32)],
    )(xv, xc, bv.astype(jnp.float32).reshape(N // BN, 1, BN),
      bc.astype(jnp.float32).reshape(N // BN, 1, BN))


def _final_body(sv_ref, sc_ref, cv_ref, cc_ref, w1_ref, b1_ref, w2_ref, b2_ref,
                out_ref):
    pred = sv_ref[...] / jnp.maximum(cv_ref[:, :1], 1.0)
    pred += sc_ref[...] / jnp.maximum(cc_ref[:, :1], 1.0)
    h = jnp.maximum(
        jnp.dot(pred, w1_ref[...], preferred_element_type=jnp.float32)
        + b1_ref[...], 0.0)
    out_ref[...] = jnp.dot(h, w2_ref[...],
                           preferred_element_type=jnp.float32) + b2_ref[...]


def _final(sv, sc, cv, cc, w1, b1, w2, b2):
    return pl.pallas_call(
        _final_body,
        out_shape=jax.ShapeDtypeStruct((G16, D), jnp.float32),
    )(sv, sc, cv, cc, w1, b1.reshape(1, D), w2, b2.reshape(1, D))


def _sc_edge(p2, meta, w_attr2):
    mesh = plsc.VectorSubcoreMesh(core_axis_name="c", subcore_axis_name="s")

    @functools.partial(
        pl.kernel,
        mesh=mesh,
        compiler_params=pltpu.CompilerParams(use_tc_tiling_on_sc=False),
        out_type=jax.ShapeDtypeStruct((2, N, SROW), jnp.float32),
        scratch_types=[
            pltpu.VMEM((4, CB), jnp.float32),
            pltpu.VMEM((4, CB), jnp.float32),
            pltpu.VMEM((CB,), jnp.int32),
            pltpu.VMEM((CB,), jnp.int32),
            pltpu.VMEM((CB,), jnp.int32),
            pltpu.VMEM((CB,), jnp.int32),
            pltpu.VMEM((CB, DH), jnp.float32),
            pltpu.VMEM((CB, DH), jnp.float32),
            pltpu.VMEM((CB, SROW), jnp.float32),
            pltpu.VMEM((DH,), jnp.float32),
            pltpu.VMEM_SHARED((N, SROW), jnp.float32),
            pltpu.SemaphoreType.DMA,
            pltpu.SemaphoreType.DMA,
            pltpu.SemaphoreType.DMA,
            pltpu.SemaphoreType.DMA,
        ],
    )
    def k(p2_h, mt_h, wa_h, out_h,
          m0, m1, g0, g1, d0, d1, r0, r1, out_v, wa_v, acc, s0, s1, s2, s3):
        cid = lax.axis_index("c")
        sid = lax.axis_index("s")
        pltpu.sync_copy(wa_h.at[cid], wa_v)
        mbuf = (m0, m1)
        gbuf = (g0, g1)
        dbuf = (d0, d1)
        rbuf = (r0, r1)
        sems = (s0, s1)
        msems = (s2, s3)
        off = cid * N
        lane = lax.iota(jnp.int32, 16)

        def fill_issue(b):
            for j in range(CB // 16):
                sl = pl.ds(16 * j, 16)
                gbuf[b][sl] = mbuf[b][0, sl].astype(jnp.int32) + off
                dbuf[b][sl] = mbuf[b][1, sl].astype(jnp.int32)
            pltpu.async_copy(p2_h.at[gbuf[b]], rbuf[b], sems[b])

        def wait_g(b):
            pltpu.make_async_copy(p2_h.at[gbuf[b]], rbuf[b], sems[b]).wait()

        def issue_meta(gch, b):
            c = sid + 16 * gch
            pltpu.async_copy(mt_h.at[c], mbuf[b], msems[b])

        def wait_meta(b):
            pltpu.make_async_copy(mt_h.at[0], mbuf[b], msems[b]).wait()

        w_regs = [wa_v[pl.ds(16 * kk, 16)] for kk in range(DH // 16)]

        def compute_scatter(b):
            def group(j, _):
                a16 = mbuf[b][2, pl.ds(16 * j, 16)]
                n16 = mbuf[b][3, pl.ds(16 * j, 16)]
                for li in range(16):
                    sel = jnp.full((16, 1), li, jnp.int32)
                    a = lax.gather(a16, sel, _GDN, (1,),
                                   mode=lax.GatherScatterMode.PROMISE_IN_BOUNDS)
                    n = lax.gather(n16, sel, _GDN, (1,),
                                   mode=lax.GatherScatterMode.PROMISE_IN_BOUNDS)
                    e = 16 * j + li
                    out_v[e, pl.ds(DH, 16)] = jnp.where(lane == 0, n, 0.0)
                    for kk in range(DH // 16):
                        v = rbuf[b][e, pl.ds(16 * kk, 16)]
                        out_v[e, pl.ds(16 * kk, 16)] = (
                            jnp.maximum(v + a * w_regs[kk], 0.0) * n)
                return 0

            lax.fori_loop(0, CB // 16, group, 0, unroll=CB // 16)
            pltpu.sync_copy(out_v, acc.at[dbuf[b]], add=True)

        pltpu.sync_copy(mt_h.at[sid], mbuf[0])
        pltpu.sync_copy(mt_h.at[sid + 16], mbuf[1])
        fill_issue(0)

        z16 = jnp.zeros((16,), jnp.float32)

        def zrow(r, _):
            for j in range(SROW // 16):
                out_v[r, pl.ds(16 * j, 16)] = z16
            return 0

        lax.fori_loop(0, CB, zrow, 0)
        rows_per_tile = N // 16
        zbase = sid * rows_per_tile
        for t in range(rows_per_tile // CB):
            pltpu.sync_copy(out_v, acc.at[pl.ds(zbase + CB * t, CB)])
        zrem = rows_per_tile % CB
        if zrem:
            pltpu.sync_copy(
                out_v.at[pl.ds(0, zrem)],
                acc.at[pl.ds(zbase + (rows_per_tile // CB) * CB, zrem)])
        plsc.subcore_barrier()

        npt = NCHUNK // 16

        fill_issue(1)
        wait_g(0)
        compute_scatter(0)
        issue_meta(2, 0)

        def body(gg, _):
            g0c = 2 * gg + 1
            wait_meta(0)
            fill_issue(0)
            wait_g(1)
            compute_scatter(1)
            nxt = jnp.minimum(g0c + 2, npt - 1)
            issue_meta(nxt, 1)
            wait_meta(1)
            fill_issue(1)
            wait_g(0)
            compute_scatter(0)
            nxt2 = jnp.minimum(g0c + 3, npt - 1)
            issue_meta(nxt2, 0)
            return 0

        lax.fori_loop(0, (npt - 1) // 2, body, 0)
        wait_meta(0)
        wait_g(1)

        plsc.subcore_barrier()
        for t in range(rows_per_tile // CB):
            rows = pl.ds(zbase + CB * t, CB)
            pltpu.sync_copy(acc.at[rows], out_h.at[cid, rows])
        if zrem:
            rows = pl.ds(zbase + (rows_per_tile // CB) * CB, zrem)
            pltpu.sync_copy(acc.at[rows], out_h.at[cid, rows])

    return k(p2, meta, w_attr2)


def kernel(b, q, edge_index_v2c, edge_attr_v2c, norm_v2c, norm_c2v,
           batch_vals, batch_cons, num_graphs, params):
    p = params
    src = edge_index_v2c[0]
    dst = edge_index_v2c[1]
    attr = edge_attr_v2c[:, 0]

    convs = p['convs']
    ms, wu1bs = [], []
    for lp in convs:
        for dname in ('v2c', 'c2v'):
            (_, _), (wm2, bm2) = lp[dname]['msg']
            (wu1, _), (_, _) = lp[dname]['upd']
            ms.append(jnp.concatenate(
                [wm2, bm2[None, :], jnp.zeros((7, D), jnp.float32)], axis=0))
            wu1bs.append(wu1[D:])
    comb = _wcomb(jnp.stack(ms), jnp.stack(wu1bs))

    def dir_params(li, dname, ci):
        (wm1, bm1), _ = convs[li][dname]['msg']
        (wu1, bu1), (wu2, bu2) = convs[li][dname]['upd']
        return dict(
            pre_w=(wm1[:D], bm1),
            w_attr2=wm1[D].reshape(2, DH),
            wu1a=wu1[:D], bu1=bu1, wu2=wu2, bu2=bu2,
            wc=comb[ci, :D], bc=comb[ci, D],
        )

    (bw1, bb1), (bw2, bb2) = p['b_enc']
    (qw1, qb1), (qw2, qb2) = p['q_enc']

    cons0 = _enc(b, bw1[0], bb1, bw2, bb2)
    l0v = dir_params(0, 'v2c', 0)
    vals0, pv = _enc(q, qw1[0], qb1, qw2, qb2, pre_w=l0v['pre_w'])

    def pack4(a0, a1, a2, a3):
        m = jnp.stack([a0.astype(jnp.float32), a1.astype(jnp.float32), a2, a3])
        return m.reshape(4, NCHUNK, CB).transpose(1, 0, 2)

    mt_v2c = pack4(src, dst, attr, norm_v2c)
    mt_c2v = pack4(dst, src, attr, norm_c2v)

    x_cons, x_vals = cons0, vals0
    for li in range(2):
        dv = dir_params(li, 'v2c', 2 * li)
        dc = dir_params(li, 'c2v', 2 * li + 1)
        ev = _sc_edge(pv.reshape(2 * N, DH), mt_v2c, dv['w_attr2'])
        x_cons, pc = _upd(x_cons, ev, cons0, dv['wu1a'], dv['wc'], dv['bc'],
                          dv['bu1'], dv['wu2'], dv['bu2'], pre_w=dc['pre_w'])
        ec = _sc_edge(pc.reshape(2 * N, DH), mt_c2v, dc['w_attr2'])
        if li == 0:
            nxt = dir_params(1, 'v2c', 2)
            x_vals, pv = _upd(x_vals, ec, vals0, dc['wu1a'], dc['wc'],
                              dc['bc'], dc['bu1'], dc['wu2'], dc['bu2'],
                              pre_w=nxt['pre_w'])
        else:
            x_vals = _upd(x_vals, ec, vals0, dc['wu1a'], dc['wc'], dc['bc'],
                          dc['bu1'], dc['wu2'], dc['bu2'])

    sv, sc_, cv, cc = _pool(x_vals, x_cons, batch_vals, batch_cons)
    (fw1, fb1), (fw2, fb2) = p['fc']
    return _final(sv, sc_, cv, cc, fw1, fb1, fw2, fb2)


# speedup vs baseline: 1.6702x; 1.0675x over previous
import functools

import jax
import jax.numpy as jnp
from jax import lax
from jax.experimental import pallas as pl
from jax.experimental.pallas import tpu as pltpu
from jax.experimental.pallas import tpu_sc as plsc

N = 10000
E = 160000
D = 256
DH = 128
SROW = 144
G16 = 16
BN = 1000
CB = 80
NCHUNK = E // CB
_GDN = lax.GatherDimensionNumbers(offset_dims=(), collapsed_slice_dims=(0,),
                                  start_index_map=(0,))


def _enc_body(with_pre, x_ref, w1_ref, b1_ref, w2_ref, b2_ref, *rest):
    if with_pre:
        wx_ref, bx_ref, out_ref, pre_ref = rest
    else:
        (out_ref,) = rest
    h = jnp.maximum(x_ref[...] * w1_ref[...] + b1_ref[...], 0.0)
    out = jnp.dot(h, w2_ref[...], preferred_element_type=jnp.float32) + b2_ref[...]
    out_ref[...] = out
    if with_pre:
        p = jnp.dot(out, wx_ref[...], preferred_element_type=jnp.float32) + bx_ref[...]
        pre_ref[0] = p[:, :DH]
        pre_ref[1] = p[:, DH:]


def _enc(x, w1, b1, w2, b2, pre_w=None):
    with_pre = pre_w is not None
    wspec = pl.BlockSpec((D, D), lambda i: (0, 0))
    vspec = pl.BlockSpec((1, D), lambda i: (0, 0))
    in_specs = [pl.BlockSpec((BN, 1), lambda i: (i, 0)), vspec, vspec, wspec, vspec]
    args = [x.reshape(N, 1), w1.reshape(1, D), b1.reshape(1, D), w2, b2.reshape(1, D)]
    out_shape = [jax.ShapeDtypeStruct((N, D), jnp.float32)]
    out_specs = [pl.BlockSpec((BN, D), lambda i: (i, 0))]
    if with_pre:
        in_specs += [wspec, vspec]
        args += [pre_w[0], pre_w[1].reshape(1, D)]
        out_shape.append(jax.ShapeDtypeStruct((2, N, DH), jnp.float32))
        out_specs.append(pl.BlockSpec((2, BN, DH), lambda i: (0, i, 0)))
    r = pl.pallas_call(
        functools.partial(_enc_body, with_pre),
        grid=(N // BN,),
        in_specs=in_specs,
        out_specs=out_specs,
        out_shape=out_shape,
    )(*args)
    return r if with_pre else r[0]


def _upd_body(with_pre, x_ref, ev_ref, x0_ref, wa_ref, wc_ref, bc_ref, b1_ref,
              w2_ref, b2_ref, *rest):
    if with_pre:
        wx_ref, bx_ref, out_ref, pre_ref = rest
    else:
        (out_ref,) = rest
    agg = jnp.concatenate([ev_ref[0, :, :DH], ev_ref[1, :, :DH]], axis=1)
    s = ev_ref[0, :, DH:DH + 1]
    h = jnp.dot(x_ref[...], wa_ref[...], preferred_element_type=jnp.float32)
    h += jnp.dot(agg, wc_ref[...], preferred_element_type=jnp.float32)
    h = jnp.maximum(h + s * bc_ref[...] + b1_ref[...], 0.0)
    out = jnp.dot(h, w2_ref[...], preferred_element_type=jnp.float32)
    out = jnp.maximum(out + b2_ref[...] + x0_ref[...], 0.0)
    out_ref[...] = out
    if with_pre:
        p = jnp.dot(out, wx_ref[...], preferred_element_type=jnp.float32) + bx_ref[...]
        pre_ref[0] = p[:, :DH]
        pre_ref[1] = p[:, DH:]


def _upd(x, ev, x0, wu1a, wc, bc, bu1, wu2, bu2, pre_w=None):
    with_pre = pre_w is not None
    wspec = pl.BlockSpec((D, D), lambda i: (0, 0))
    vspec = pl.BlockSpec((1, D), lambda i: (0, 0))
    nspec = pl.BlockSpec((BN, D), lambda i: (i, 0))
    in_specs = [nspec, pl.BlockSpec((2, BN, SROW), lambda i: (0, i, 0)), nspec,
                wspec, wspec, vspec, vspec, wspec, vspec]
    args = [x, ev, x0, wu1a, wc, bc.reshape(1, D), bu1.reshape(1, D), wu2,
            bu2.reshape(1, D)]
    out_shape = [jax.ShapeDtypeStruct((N, D), jnp.float32)]
    out_specs = [nspec]
    if with_pre:
        in_specs += [wspec, vspec]
        args += [pre_w[0], pre_w[1].reshape(1, D)]
        out_shape.append(jax.ShapeDtypeStruct((2, N, DH), jnp.float32))
        out_specs.append(pl.BlockSpec((2, BN, DH), lambda i: (0, i, 0)))
    r = pl.pallas_call(
        functools.partial(_upd_body, with_pre),
        grid=(N // BN,),
        in_specs=in_specs,
        out_specs=out_specs,
        out_shape=out_shape,
    )(*args)
    return r if with_pre else r[0]


def _wcomb_body(m_ref, w_ref, out_ref):
    out_ref[0] = jnp.dot(m_ref[0], w_ref[0], preferred_element_type=jnp.float32)


def _wcomb(ms, wu1bs):
    return pl.pallas_call(
        _wcomb_body,
        grid=(4,),
        in_specs=[pl.BlockSpec((1, 264, D), lambda i: (i, 0, 0)),
                  pl.BlockSpec((1, D, D), lambda i: (i, 0, 0))],
        out_specs=pl.BlockSpec((1, 264, D), lambda i: (i, 0, 0)),
        out_shape=jax.ShapeDtypeStruct((4, 264, D), jnp.float32),
    )(ms, wu1bs)


def _pool_body(xv_ref, xc_ref, bv_ref, bc_ref, sv_ref, sc_ref, cv_ref, cc_ref):
    i = pl.program_id(0)
    gi = lax.broadcasted_iota(jnp.int32, (G16, BN), 0).astype(jnp.float32)
    mv = (gi == bv_ref[0]).astype(jnp.float32)
    mc = (gi == bc_ref[0]).astype(jnp.float32)
    pv = jnp.dot(mv, xv_ref[...], preferred_element_type=jnp.float32)
    pc = jnp.dot(mc, xc_ref[...], preferred_element_type=jnp.float32)
    cv = jnp.broadcast_to(jnp.sum(mv, axis=1, keepdims=True), (G16, 128))
    cc = jnp.broadcast_to(jnp.sum(mc, axis=1, keepdims=True), (G16, 128))

    @pl.when(i == 0)
    def _():
        sv_ref[...] = pv
        sc_ref[...] = pc
        cv_ref[...] = cv
        cc_ref[...] = cc

    @pl.when(i > 0)
    def _():
        sv_ref[...] += pv
        sc_ref[...] += pc
        cv_ref[...] += cv
        cc_ref[...] += cc


def _pool(xv, xc, bv, bc):
    bspec = pl.BlockSpec((1, 1, BN), lambda i: (i, 0, 0))
    nspec = pl.BlockSpec((BN, D), lambda i: (i, 0))
    sspec = pl.BlockSpec((G16, D), lambda i: (0, 0))
    cspec = pl.BlockSpec((G16, 128), lambda i: (0, 0))
    return pl.pallas_call(
        _pool_body,
        grid=(N // BN,),
        in_specs=[nspec, nspec, bspec, bspec],
        out_specs=[sspec, sspec, cspec, cspec],
        out_shape=[jax.ShapeDtypeStruct((G16, D), jnp.float32),
                   jax.ShapeDtypeStruct((G16, D), jnp.float32),
                   jax.ShapeDtypeStruct((G16, 128), jnp.float32),
                   jax.ShapeDtypeStruct((G16, 128), jnp.float32)],
    )(xv, xc, bv.astype(jnp.float32).reshape(N // BN, 1, BN),
      bc.astype(jnp.float32).reshape(N // BN, 1, BN))


def _final_body(sv_ref, sc_ref, cv_ref, cc_ref, w1_ref, b1_ref, w2_ref, b2_ref,
                out_ref):
    pred = sv_ref[...] / jnp.maximum(cv_ref[:, :1], 1.0)
    pred += sc_ref[...] / jnp.maximum(cc_ref[:, :1], 1.0)
    h = jnp.maximum(
        jnp.dot(pred, w1_ref[...], preferred_element_type=jnp.float32)
        + b1_ref[...], 0.0)
    out_ref[...] = jnp.dot(h, w2_ref[...],
                           preferred_element_type=jnp.float32) + b2_ref[...]


def _final(sv, sc, cv, cc, w1, b1, w2, b2):
    return pl.pallas_call(
        _final_body,
        out_shape=jax.ShapeDtypeStruct((G16, D), jnp.float32),
    )(sv, sc, cv, cc, w1, b1.reshape(1, D), w2, b2.reshape(1, D))


def _sc_edge(p2, meta, w_attr2):
    mesh = plsc.VectorSubcoreMesh(core_axis_name="c", subcore_axis_name="s")

    @functools.partial(
        pl.kernel,
        mesh=mesh,
        compiler_params=pltpu.CompilerParams(use_tc_tiling_on_sc=False),
        out_type=jax.ShapeDtypeStruct((2, N, SROW), jnp.float32),
        scratch_types=[
            pltpu.VMEM((4, CB), jnp.float32),
            pltpu.VMEM((4, CB), jnp.float32),
            pltpu.VMEM((CB,), jnp.int32),
            pltpu.VMEM((CB,), jnp.int32),
            pltpu.VMEM((CB,), jnp.int32),
            pltpu.VMEM((CB,), jnp.int32),
            pltpu.VMEM((CB, DH), jnp.float32),
            pltpu.VMEM((CB, DH), jnp.float32),
            pltpu.VMEM((CB, SROW), jnp.float32),
            pltpu.VMEM((DH,), jnp.float32),
            pltpu.VMEM_SHARED((N, SROW), jnp.float32),
            pltpu.SemaphoreType.DMA,
            pltpu.SemaphoreType.DMA,
            pltpu.SemaphoreType.DMA,
            pltpu.SemaphoreType.DMA,
        ],
    )
    def k(p2_h, mt_h, wa_h, out_h,
          m0, m1, g0, g1, d0, d1, r0, r1, out_v, wa_v, acc, s0, s1, s2, s3):
        cid = lax.axis_index("c")
        sid = lax.axis_index("s")
        pltpu.sync_copy(wa_h.at[cid], wa_v)
        mbuf = (m0, m1)
        gbuf = (g0, g1)
        dbuf = (d0, d1)
        rbuf = (r0, r1)
        sems = (s0, s1)
        msems = (s2, s3)
        off = cid * N
        lane = lax.iota(jnp.int32, 16)

        def fill_issue(b):
            for j in range(CB // 16):
                sl = pl.ds(16 * j, 16)
                gbuf[b][sl] = mbuf[b][0, sl].astype(jnp.int32) + off
                dbuf[b][sl] = mbuf[b][1, sl].astype(jnp.int32)
            pltpu.async_copy(p2_h.at[gbuf[b]], rbuf[b], sems[b])

        def wait_g(b):
            pltpu.make_async_copy(p2_h.at[gbuf[b]], rbuf[b], sems[b]).wait()

        def issue_meta(gch, b):
            c = sid + 16 * gch
            pltpu.async_copy(mt_h.at[c], mbuf[b], msems[b])

        def wait_meta(b):
            pltpu.make_async_copy(mt_h.at[0], mbuf[b], msems[b]).wait()

        w_regs = [wa_v[pl.ds(16 * kk, 16)] for kk in range(DH // 16)]

        def compute_scatter(b):
            def group(j, _):
                a16 = mbuf[b][2, pl.ds(16 * j, 16)]
                n16 = mbuf[b][3, pl.ds(16 * j, 16)]
                for li in range(16):
                    sel = jnp.full((16, 1), li, jnp.int32)
                    a = lax.gather(a16, sel, _GDN, (1,),
                                   mode=lax.GatherScatterMode.PROMISE_IN_BOUNDS)
                    n = lax.gather(n16, sel, _GDN, (1,),
                                   mode=lax.GatherScatterMode.PROMISE_IN_BOUNDS)
                    e = 16 * j + li
                    out_v[e, pl.ds(DH, 16)] = jnp.where(lane == 0, n, 0.0)
                    vs = [rbuf[b][e, pl.ds(16 * kk, 16)]
                          for kk in range(DH // 16)]
                    for kk in range(DH // 16):
                        out_v[e, pl.ds(16 * kk, 16)] = (
                            jnp.maximum(vs[kk] + a * w_regs[kk], 0.0) * n)
                return 0

            lax.fori_loop(0, CB // 16, group, 0, unroll=CB // 16)
            pltpu.sync_copy(out_v, acc.at[dbuf[b]], add=True)

        pltpu.sync_copy(mt_h.at[sid], mbuf[0])
        pltpu.sync_copy(mt_h.at[sid + 16], mbuf[1])
        fill_issue(0)

        z16 = jnp.zeros((16,), jnp.float32)

        def zrow(r, _):
            for j in range(SROW // 16):
                out_v[r, pl.ds(16 * j, 16)] = z16
            return 0

        lax.fori_loop(0, CB, zrow, 0)
        rows_per_tile = N // 16
        zbase = sid * rows_per_tile
        for t in range(rows_per_tile // CB):
            pltpu.sync_copy(out_v, acc.at[pl.ds(zbase + CB * t, CB)])
        zrem = rows_per_tile % CB
        if zrem:
            pltpu.sync_copy(
                out_v.at[pl.ds(0, zrem)],
                acc.at[pl.ds(zbase + (rows_per_tile // CB) * CB, zrem)])
        plsc.subcore_barrier()

        npt = NCHUNK // 16

        fill_issue(1)
        wait_g(0)
        compute_scatter(0)
        issue_meta(2, 0)

        def body(gg, _):
            g0c = 2 * gg + 1
            wait_meta(0)
            fill_issue(0)
            wait_g(1)
            compute_scatter(1)
            nxt = jnp.minimum(g0c + 2, npt - 1)
            issue_meta(nxt, 1)
            wait_meta(1)
            fill_issue(1)
            wait_g(0)
            compute_scatter(0)
            nxt2 = jnp.minimum(g0c + 3, npt - 1)
            issue_meta(nxt2, 0)
            return 0

        lax.fori_loop(0, (npt - 1) // 2, body, 0)
        wait_meta(0)
        wait_g(1)

        plsc.subcore_barrier()
        for t in range(rows_per_tile // CB):
            rows = pl.ds(zbase + CB * t, CB)
            pltpu.sync_copy(acc.at[rows], out_h.at[cid, rows])
        if zrem:
            rows = pl.ds(zbase + (rows_per_tile // CB) * CB, zrem)
            pltpu.sync_copy(acc.at[rows], out_h.at[cid, rows])

    return k(p2, meta, w_attr2)


def kernel(b, q, edge_index_v2c, edge_attr_v2c, norm_v2c, norm_c2v,
           batch_vals, batch_cons, num_graphs, params):
    p = params
    src = edge_index_v2c[0]
    dst = edge_index_v2c[1]
    attr = edge_attr_v2c[:, 0]

    convs = p['convs']
    ms, wu1bs = [], []
    for lp in convs:
        for dname in ('v2c', 'c2v'):
            (_, _), (wm2, bm2) = lp[dname]['msg']
            (wu1, _), (_, _) = lp[dname]['upd']
            ms.append(jnp.concatenate(
                [wm2, bm2[None, :], jnp.zeros((7, D), jnp.float32)], axis=0))
            wu1bs.append(wu1[D:])
    comb = _wcomb(jnp.stack(ms), jnp.stack(wu1bs))

    def dir_params(li, dname, ci):
        (wm1, bm1), _ = convs[li][dname]['msg']
        (wu1, bu1), (wu2, bu2) = convs[li][dname]['upd']
        return dict(
            pre_w=(wm1[:D], bm1),
            w_attr2=wm1[D].reshape(2, DH),
            wu1a=wu1[:D], bu1=bu1, wu2=wu2, bu2=bu2,
            wc=comb[ci, :D], bc=comb[ci, D],
        )

    (bw1, bb1), (bw2, bb2) = p['b_enc']
    (qw1, qb1), (qw2, qb2) = p['q_enc']

    cons0 = _enc(b, bw1[0], bb1, bw2, bb2)
    l0v = dir_params(0, 'v2c', 0)
    vals0, pv = _enc(q, qw1[0], qb1, qw2, qb2, pre_w=l0v['pre_w'])

    def pack4(a0, a1, a2, a3):
        m = jnp.stack([a0.astype(jnp.float32), a1.astype(jnp.float32), a2, a3])
        return m.reshape(4, NCHUNK, CB).transpose(1, 0, 2)

    mt_v2c = pack4(src, dst, attr, norm_v2c)
    mt_c2v = pack4(dst, src, attr, norm_c2v)

    x_cons, x_vals = cons0, vals0
    for li in range(2):
        dv = dir_params(li, 'v2c', 2 * li)
        dc = dir_params(li, 'c2v', 2 * li + 1)
        ev = _sc_edge(pv.reshape(2 * N, DH), mt_v2c, dv['w_attr2'])
        x_cons, pc = _upd(x_cons, ev, cons0, dv['wu1a'], dv['wc'], dv['bc'],
                          dv['bu1'], dv['wu2'], dv['bu2'], pre_w=dc['pre_w'])
        ec = _sc_edge(pc.reshape(2 * N, DH), mt_c2v, dc['w_attr2'])
        if li == 0:
            nxt = dir_params(1, 'v2c', 2)
            x_vals, pv = _upd(x_vals, ec, vals0, dc['wu1a'], dc['wc'],
                              dc['bc'], dc['bu1'], dc['wu2'], dc['bu2'],
                              pre_w=nxt['pre_w'])
        else:
            x_vals = _upd(x_vals, ec, vals0, dc['wu1a'], dc['wc'], dc['bc'],
                          dc['bu1'], dc['wu2'], dc['bu2'])

    sv, sc_, cv, cc = _pool(x_vals, x_cons, batch_vals, batch_cons)
    (fw1, fb1), (fw2, fb2) = p['fc']
    return _final(sv, sc_, cv, cc, fw1, fb1, fw2, fb2)
